# baseline (speedup 1.0000x reference)
import functools

import jax
import jax.numpy as jnp
from jax import lax
from jax.experimental import pallas as pl
from jax.experimental.pallas import tpu as pltpu
from jax.experimental.pallas import tpu_sc as plsc

BATCH = 16384
LANES = 16


@functools.cache
def _build(num_cores: int, num_subcores: int):
    nw = num_cores * num_subcores
    b_per_w = BATCH // nw
    n_chunks = 4
    chunk = b_per_w // n_chunks
    mesh = plsc.VectorSubcoreMesh(core_axis_name="c", subcore_axis_name="s")

    @functools.partial(
        pl.kernel,
        mesh=mesh,
        out_type=jax.ShapeDtypeStruct((BATCH,), jnp.float32),
        scratch_types=[
            pltpu.VMEM((b_per_w,), jnp.int32),
            pltpu.VMEM((b_per_w,), jnp.float32),
            pltpu.VMEM((2 * LANES,), jnp.float32),
            pltpu.SemaphoreType.DMA,
            pltpu.SemaphoreType.DMA,
        ]
        + [pltpu.SemaphoreType.DMA for _ in range(n_chunks)]
        + [pltpu.SemaphoreType.DMA],
    )
    def gather_kernel(idx_hbm, table_hbm, mn_hbm, out_hbm,
                      idx_v, vals_v, mn_v, sem_i, sem_m,
                      *gather_store_sems):
        g_sems = gather_store_sems[:n_chunks]
        sem_s = gather_store_sems[n_chunks]
        wid = lax.axis_index("s") * num_cores + lax.axis_index("c")
        base = wid * b_per_w
        cp_idx = pltpu.async_copy(idx_hbm.at[pl.ds(base, b_per_w)], idx_v,
                                  sem_i)
        cp_mn = pltpu.async_copy(mn_hbm, mn_v, sem_m)
        cp_idx.wait()
        gathers = []
        for k in range(n_chunks):
            sl = pl.ds(k * chunk, chunk)
            gathers.append(pltpu.async_copy(table_hbm.at[idx_v.at[sl]],
                                            vals_v.at[sl], g_sems[k]))
        cp_mn.wait()
        neg_mean = -mn_v[pl.ds(0, LANES)]
        inv_norm = 1.0 / mn_v[pl.ds(LANES, LANES)]
        stores = []
        for k in range(n_chunks):
            gathers[k].wait()

            def body(i, carry, k=k):
                sl = pl.ds(k * chunk + i * LANES, LANES)
                vals_v[sl] = (vals_v[sl] + neg_mean) * inv_norm
                return carry

            lax.fori_loop(0, chunk // LANES, body, 0)
            sl = pl.ds(k * chunk, chunk)
            stores.append(pltpu.async_copy(
                vals_v.at[sl], out_hbm.at[pl.ds(base + k * chunk, chunk)],
                sem_s))
        for st in stores:
            st.wait()

    return gather_kernel


def kernel(inputs, categ_bias, moving_mean, moving_norm):
    idx = inputs.reshape(BATCH).astype(jnp.int32)
    table = categ_bias.reshape(-1).astype(jnp.float32)
    mn = jnp.concatenate([
        jnp.broadcast_to(moving_mean.reshape(1), (LANES,)),
        jnp.broadcast_to(moving_norm.reshape(1), (LANES,)),
    ]).astype(jnp.float32)
    info = plsc.get_sparse_core_info()
    out = _build(info.num_cores, info.num_subcores)(idx, table, mn)
    return out.reshape(BATCH, 1)


# speedup vs baseline: 1.0057x; 1.0057x over previous
import functools

import jax
import jax.numpy as jnp
from jax import lax
from jax.experimental import pallas as pl
from jax.experimental.pallas import tpu as pltpu
from jax.experimental.pallas import tpu_sc as plsc

BATCH = 16384
LANES = 16


@functools.cache
def _build(num_cores: int, num_subcores: int):
    nw = num_cores * num_subcores
    b_per_w = BATCH // nw
    n_chunks = 2
    chunk = b_per_w // n_chunks
    mesh = plsc.VectorSubcoreMesh(core_axis_name="c", subcore_axis_name="s")

    @functools.partial(
        pl.kernel,
        mesh=mesh,
        out_type=jax.ShapeDtypeStruct((BATCH,), jnp.float32),
        scratch_types=[
            pltpu.VMEM((b_per_w,), jnp.int32),
            pltpu.VMEM((b_per_w,), jnp.float32),
            pltpu.VMEM((2 * LANES,), jnp.float32),
            pltpu.SemaphoreType.DMA,
            pltpu.SemaphoreType.DMA,
        ]
        + [pltpu.SemaphoreType.DMA for _ in range(n_chunks)]
        + [pltpu.SemaphoreType.DMA],
    )
    def gather_kernel(idx_hbm, table_hbm, mn_hbm, out_hbm,
                      idx_v, vals_v, mn_v, sem_i, sem_m,
                      *gather_store_sems):
        g_sems = gather_store_sems[:n_chunks]
        sem_s = gather_store_sems[n_chunks]
        wid = lax.axis_index("s") * num_cores + lax.axis_index("c")
        base = wid * b_per_w
        cp_idx = pltpu.async_copy(idx_hbm.at[pl.ds(base, b_per_w)], idx_v,
                                  sem_i)
        cp_mn = pltpu.async_copy(mn_hbm, mn_v, sem_m)
        cp_idx.wait()
        gathers = []
        for k in range(n_chunks):
            sl = pl.ds(k * chunk, chunk)
            gathers.append(pltpu.async_copy(table_hbm.at[idx_v.at[sl]],
                                            vals_v.at[sl], g_sems[k]))
        cp_mn.wait()
        neg_mean = -mn_v[pl.ds(0, LANES)]
        inv_norm = 1.0 / mn_v[pl.ds(LANES, LANES)]
        stores = []
        for k in range(n_chunks):
            gathers[k].wait()

            def body(i, carry, k=k):
                sl = pl.ds(k * chunk + i * LANES, LANES)
                vals_v[sl] = (vals_v[sl] + neg_mean) * inv_norm
                return carry

            lax.fori_loop(0, chunk // LANES, body, 0)
            sl = pl.ds(k * chunk, chunk)
            stores.append(pltpu.async_copy(
                vals_v.at[sl], out_hbm.at[pl.ds(base + k * chunk, chunk)],
                sem_s))
        for st in stores:
            st.wait()

    return gather_kernel


def kernel(inputs, categ_bias, moving_mean, moving_norm):
    idx = inputs.reshape(BATCH).astype(jnp.int32)
    table = categ_bias.reshape(-1).astype(jnp.float32)
    mn = jnp.concatenate([
        jnp.broadcast_to(moving_mean.reshape(1), (LANES,)),
        jnp.broadcast_to(moving_norm.reshape(1), (LANES,)),
    ]).astype(jnp.float32)
    info = plsc.get_sparse_core_info()
    out = _build(info.num_cores, info.num_subcores)(idx, table, mn)
    return out.reshape(BATCH, 1)


# speedup vs baseline: 1.0940x; 1.0878x over previous
import functools

import jax
import jax.numpy as jnp
from jax import lax
from jax.experimental import pallas as pl
from jax.experimental.pallas import tpu as pltpu
from jax.experimental.pallas import tpu_sc as plsc

BATCH = 16384
LANES = 16


@functools.cache
def _build(num_cores: int, num_subcores: int):
    nw = num_cores * num_subcores
    b_per_w = BATCH // nw
    n_chunks = 2
    chunk = b_per_w // n_chunks
    mesh = plsc.VectorSubcoreMesh(core_axis_name="c", subcore_axis_name="s")

    @functools.partial(
        pl.kernel,
        mesh=mesh,
        out_type=jax.ShapeDtypeStruct((BATCH,), jnp.float32),
        scratch_types=[
            pltpu.VMEM((b_per_w,), jnp.int32),
            pltpu.VMEM((b_per_w,), jnp.float32),
            pltpu.VMEM((LANES,), jnp.float32),
            pltpu.VMEM((LANES,), jnp.float32),
            pltpu.SemaphoreType.DMA,
            pltpu.SemaphoreType.DMA,
        ]
        + [pltpu.SemaphoreType.DMA for _ in range(n_chunks)]
        + [pltpu.SemaphoreType.DMA],
    )
    def gather_kernel(idx_hbm, table_hbm, mean_hbm, norm_hbm, out_hbm,
                      idx_v, vals_v, mean_v, norm_v, sem_i, sem_m,
                      *gather_store_sems):
        g_sems = gather_store_sems[:n_chunks]
        sem_s = gather_store_sems[n_chunks]
        wid = lax.axis_index("s") * num_cores + lax.axis_index("c")
        base = wid * b_per_w
        cp_idx = pltpu.async_copy(idx_hbm.at[pl.ds(base, b_per_w)], idx_v,
                                  sem_i)
        cp_mean = pltpu.async_copy(mean_hbm, mean_v.at[pl.ds(0, 1)], sem_m)
        cp_norm = pltpu.async_copy(norm_hbm, norm_v.at[pl.ds(0, 1)], sem_m)
        cp_idx.wait()
        gathers = []
        for k in range(n_chunks):
            sl = pl.ds(k * chunk, chunk)
            gathers.append(pltpu.async_copy(table_hbm.at[idx_v.at[sl]],
                                            vals_v.at[sl], g_sems[k]))
        cp_mean.wait()
        cp_norm.wait()
        m = mean_v[...][0]
        n = norm_v[...][0]
        neg_mean = jnp.full((LANES,), 0.0, jnp.float32) - m
        inv_norm = 1.0 / (jnp.full((LANES,), 0.0, jnp.float32) + n)
        stores = []
        for k in range(n_chunks):
            gathers[k].wait()

            def body(i, carry, k=k):
                sl = pl.ds(k * chunk + i * LANES, LANES)
                vals_v[sl] = (vals_v[sl] + neg_mean) * inv_norm
                return carry

            lax.fori_loop(0, chunk // LANES, body, 0)
            sl = pl.ds(k * chunk, chunk)
            stores.append(pltpu.async_copy(
                vals_v.at[sl], out_hbm.at[pl.ds(base + k * chunk, chunk)],
                sem_s))
        for st in stores:
            st.wait()

    return gather_kernel


def kernel(inputs, categ_bias, moving_mean, moving_norm):
    idx = inputs.reshape(BATCH).astype(jnp.int32)
    table = categ_bias.reshape(-1).astype(jnp.float32)
    mean = moving_mean.reshape(1).astype(jnp.float32)
    norm = moving_norm.reshape(1).astype(jnp.float32)
    info = plsc.get_sparse_core_info()
    out = _build(info.num_cores, info.num_subcores)(idx, table, mean, norm)
    return out.reshape(BATCH, 1)


# speedup vs baseline: 1.0994x; 1.0049x over previous
import functools

import jax
import jax.numpy as jnp
from jax import lax
from jax.experimental import pallas as pl
from jax.experimental.pallas import tpu as pltpu
from jax.experimental.pallas import tpu_sc as plsc

BATCH = 16384
LANES = 16


@functools.cache
def _build(num_cores: int, num_subcores: int):
    nw = num_cores * num_subcores
    b_per_w = BATCH // nw
    n_chunks = 4
    chunk = b_per_w // n_chunks
    mesh = plsc.VectorSubcoreMesh(core_axis_name="c", subcore_axis_name="s")

    @functools.partial(
        pl.kernel,
        mesh=mesh,
        out_type=jax.ShapeDtypeStruct((BATCH,), jnp.float32),
        scratch_types=[
            pltpu.VMEM((b_per_w,), jnp.int32),
            pltpu.VMEM((b_per_w,), jnp.float32),
            pltpu.VMEM((LANES,), jnp.float32),
            pltpu.VMEM((LANES,), jnp.float32),
            pltpu.SemaphoreType.DMA,
            pltpu.SemaphoreType.DMA,
        ]
        + [pltpu.SemaphoreType.DMA for _ in range(n_chunks)]
        + [pltpu.SemaphoreType.DMA],
    )
    def gather_kernel(idx_hbm, table_hbm, mean_hbm, norm_hbm, out_hbm,
                      idx_v, vals_v, mean_v, norm_v, sem_i, sem_m,
                      *gather_store_sems):
        g_sems = gather_store_sems[:n_chunks]
        sem_s = gather_store_sems[n_chunks]
        wid = lax.axis_index("s") * num_cores + lax.axis_index("c")
        base = wid * b_per_w
        cp_idx = pltpu.async_copy(idx_hbm.at[pl.ds(base, b_per_w)], idx_v,
                                  sem_i)
        cp_mean = pltpu.async_copy(mean_hbm, mean_v.at[pl.ds(0, 1)], sem_m)
        cp_norm = pltpu.async_copy(norm_hbm, norm_v.at[pl.ds(0, 1)], sem_m)
        cp_idx.wait()
        gathers = []
        for k in range(n_chunks):
            sl = pl.ds(k * chunk, chunk)
            gathers.append(pltpu.async_copy(table_hbm.at[idx_v.at[sl]],
                                            vals_v.at[sl], g_sems[k]))
        cp_mean.wait()
        cp_norm.wait()
        m = mean_v[...][0]
        n = norm_v[...][0]
        neg_mean = jnp.full((LANES,), 0.0, jnp.float32) - m
        inv_norm = 1.0 / (jnp.full((LANES,), 0.0, jnp.float32) + n)
        stores = []
        for k in range(n_chunks):
            gathers[k].wait()

            def body(i, carry, k=k):
                sl = pl.ds(k * chunk + i * LANES, LANES)
                vals_v[sl] = (vals_v[sl] + neg_mean) * inv_norm
                return carry

            lax.fori_loop(0, chunk // LANES, body, 0)
            sl = pl.ds(k * chunk, chunk)
            stores.append(pltpu.async_copy(
                vals_v.at[sl], out_hbm.at[pl.ds(base + k * chunk, chunk)],
                sem_s))
        for st in stores:
            st.wait()

    return gather_kernel


def kernel(inputs, categ_bias, moving_mean, moving_norm):
    idx = inputs.reshape(BATCH).astype(jnp.int32)
    table = categ_bias.reshape(-1).astype(jnp.float32)
    mean = moving_mean.reshape(1).astype(jnp.float32)
    norm = moving_norm.reshape(1).astype(jnp.float32)
    info = plsc.get_sparse_core_info()
    out = _build(info.num_cores, info.num_subcores)(idx, table, mean, norm)
    return out.reshape(BATCH, 1)
